# baseline (speedup 1.0000x reference)
import functools

import jax
import jax.numpy as jnp
from jax import lax
from jax.experimental import pallas as pl
from jax.experimental.pallas import tpu as pltpu
from jax.experimental.pallas import tpu_sc as plsc

ENT_DIM = 64
B_TOTAL = 16384
MARGIN_C = 2.0
ALPHA_C = 0.01

NC = 2
NS = 16
L = 16
NW = NC * NS
PER_W = B_TOTAL // NW
CHUNK = 128
NCHUNK = PER_W // CHUNK
NGROUP = CHUNK // L
NPART = 7


def _body(ent_ref, rel_ref, ph_ref, pr_ref, pt_ref, nh_ref, nr_ref, nt_ref,
          pos_out, neg_out, part_out,
          ph_i, pr_i, pt_i, nh_i, nr_i, nt_i,
          hp, rp, tp, hn, rn, tn,
          pos_s, neg_s, part_v, sem):
    wid = lax.axis_index("s") * NC + lax.axis_index("c")
    base = wid * PER_W

    iota = lax.iota(jnp.int32, L)
    zero = jnp.zeros((L,), jnp.float32)
    accs = (zero,) * NPART

    for c in range(NCHUNK):
        off = base + c * CHUNK
        pltpu.sync_copy(ph_ref.at[pl.ds(off, CHUNK)], ph_i)
        pltpu.sync_copy(pr_ref.at[pl.ds(off, CHUNK)], pr_i)
        pltpu.sync_copy(pt_ref.at[pl.ds(off, CHUNK)], pt_i)
        pltpu.sync_copy(nh_ref.at[pl.ds(off, CHUNK)], nh_i)
        pltpu.sync_copy(nr_ref.at[pl.ds(off, CHUNK)], nr_i)
        pltpu.sync_copy(nt_ref.at[pl.ds(off, CHUNK)], nt_i)

        copies = [
            pltpu.async_copy(ent_ref.at[ph_i], hp, sem),
            pltpu.async_copy(rel_ref.at[pr_i], rp, sem),
            pltpu.async_copy(ent_ref.at[pt_i], tp, sem),
            pltpu.async_copy(ent_ref.at[nh_i], hn, sem),
            pltpu.async_copy(rel_ref.at[nr_i], rn, sem),
            pltpu.async_copy(ent_ref.at[nt_i], tn, sem),
        ]
        for cp in copies:
            cp.wait()

        for g in range(NGROUP):
            rows = iota + (g * L)

            def d_step(d, carry):
                sp, sn, l1hp, r2p, l1tp, l1hn, r2n, l1tn = carry
                cols = jnp.full((L,), 0, jnp.int32) + d
                hpv = plsc.load_gather(hp, [rows, cols])
                rpv = plsc.load_gather(rp, [rows, cols])
                tpv = plsc.load_gather(tp, [rows, cols])
                hnv = plsc.load_gather(hn, [rows, cols])
                rnv = plsc.load_gather(rn, [rows, cols])
                tnv = plsc.load_gather(tn, [rows, cols])
                sp = sp + jnp.abs(hpv + rpv - tpv)
                sn = sn + jnp.abs(hnv + rnv - tnv)
                l1hp = l1hp + jnp.abs(hpv)
                r2p = r2p + rpv * rpv
                l1tp = l1tp + jnp.abs(tpv)
                l1hn = l1hn + jnp.abs(hnv)
                r2n = r2n + rnv * rnv
                l1tn = l1tn + jnp.abs(tnv)
                return (sp, sn, l1hp, r2p, l1tp, l1hn, r2n, l1tn)

            out = lax.fori_loop(0, ENT_DIM, d_step, (zero, zero) + accs[:6])
            sp, sn = out[0], out[1]
            hinge = accs[6] + jnp.maximum(0.0, sp - sn + MARGIN_C)
            accs = out[2:8] + (hinge,)
            pos_s[pl.ds(c * CHUNK + g * L, L)] = sp
            neg_s[pl.ds(c * CHUNK + g * L, L)] = sn

    for i in range(NPART):
        part_v[pl.ds(i * L, L)] = accs[i]
    part_v[pl.ds(NPART * L, L)] = zero

    pltpu.sync_copy(pos_s, pos_out.at[pl.ds(base, PER_W)])
    pltpu.sync_copy(neg_s, neg_out.at[pl.ds(base, PER_W)])
    pltpu.sync_copy(part_v, part_out.at[wid])


_sc_call = pl.kernel(
    _body,
    out_type=[
        jax.ShapeDtypeStruct((B_TOTAL,), jnp.float32),
        jax.ShapeDtypeStruct((B_TOTAL,), jnp.float32),
        jax.ShapeDtypeStruct((NW, (NPART + 1) * L), jnp.float32),
    ],
    mesh=plsc.VectorSubcoreMesh(
        core_axis_name="c", subcore_axis_name="s",
        num_cores=NC, num_subcores=NS),
    compiler_params=pltpu.CompilerParams(
        needs_layout_passes=False, use_tc_tiling_on_sc=False),
    scratch_types=[
        pltpu.VMEM((CHUNK,), jnp.int32),
        pltpu.VMEM((CHUNK,), jnp.int32),
        pltpu.VMEM((CHUNK,), jnp.int32),
        pltpu.VMEM((CHUNK,), jnp.int32),
        pltpu.VMEM((CHUNK,), jnp.int32),
        pltpu.VMEM((CHUNK,), jnp.int32),
        pltpu.VMEM((CHUNK, ENT_DIM), jnp.float32),
        pltpu.VMEM((CHUNK, ENT_DIM), jnp.float32),
        pltpu.VMEM((CHUNK, ENT_DIM), jnp.float32),
        pltpu.VMEM((CHUNK, ENT_DIM), jnp.float32),
        pltpu.VMEM((CHUNK, ENT_DIM), jnp.float32),
        pltpu.VMEM((CHUNK, ENT_DIM), jnp.float32),
        pltpu.VMEM((PER_W,), jnp.float32),
        pltpu.VMEM((PER_W,), jnp.float32),
        pltpu.VMEM(((NPART + 1) * L,), jnp.float32),
        pltpu.SemaphoreType.DMA,
    ],
)


def kernel(ent_emb, rel_emb, ph_idx, pr_idx, pt_idx, nh_idx, nr_idx, nt_idx):
    pos_scores, neg_scores, parts = _sc_call(
        ent_emb, rel_emb, ph_idx, pr_idx, pt_idx, nh_idx, nr_idx, nt_idx)
    p = jnp.sum(parts.reshape(NW, NPART + 1, L), axis=(0, 2))
    b = jnp.float32(B_TOTAL)
    pos_norms = ((p[0] / b - 1.0) + p[1] / (b * ENT_DIM) + (p[2] / b - 1.0)) / 3.0
    neg_norms = ((p[3] / b - 1.0) + p[4] / (b * ENT_DIM) + (p[5] / b - 1.0)) / 3.0
    loss = p[6] / b + ALPHA_C * pos_norms + ALPHA_C * neg_norms
    return (loss, pos_scores, neg_scores)
